# baseline (speedup 1.0000x reference)
import functools

import jax
import jax.numpy as jnp
from jax import lax
from jax.experimental import pallas as pl
from jax.experimental.pallas import tpu as pltpu
from jax.experimental.pallas import tpu_sc as plsc

_N = 100000
_D = 128
_NSEG = 2048

_ROWS = 4096
_K1 = 12 * _ROWS
_K2 = _N - _K1

_NW = 32
_L = 16


def _mlp_body(e_ref, w1_ref, b1_ref, w2_ref, b2_ref, out_ref):
    h = jnp.dot(e_ref[...], w1_ref[...], preferred_element_type=jnp.float32)
    h = h + b1_ref[...]
    h = h * jax.nn.sigmoid(h)
    s2 = lax.dot_general(
        w2_ref[...], h, (((1,), (1,)), ((), ())),
        preferred_element_type=jnp.float32,
    )
    out_ref[...] = s2[0] + b2_ref[0, 0]


def _mlp(energy, W1, b1, W2, b2, n_rows, block_off):
    grid = -(-n_rows // _ROWS)
    return pl.pallas_call(
        _mlp_body,
        grid=(grid,),
        in_specs=[
            pl.BlockSpec((_ROWS, _D), lambda i: (block_off + i, 0)),
            pl.BlockSpec((_D, _D), lambda i: (0, 0)),
            pl.BlockSpec((1, _D), lambda i: (0, 0)),
            pl.BlockSpec((1, _D), lambda i: (0, 0)),
            pl.BlockSpec((1, 1), lambda i: (0, 0)),
        ],
        out_specs=pl.BlockSpec((_ROWS,), lambda i: (i,)),
        out_shape=jax.ShapeDtypeStruct((n_rows,), jnp.float32),
    )(energy, W1, b1.reshape(1, _D), W2.reshape(1, _D), b2.reshape(1, 1))


_sc_mesh = plsc.VectorSubcoreMesh(core_axis_name="c", subcore_axis_name="s")


def _make_segsum(chunk, last, batch_off):
    @functools.partial(
        pl.kernel,
        mesh=_sc_mesh,
        compiler_params=pltpu.CompilerParams(needs_layout_passes=False),
        out_type=jax.ShapeDtypeStruct((2, _NSEG), jnp.float32),
        scratch_types=[
            pltpu.VMEM((chunk,), jnp.float32),
            pltpu.VMEM((chunk,), jnp.int32),
            pltpu.VMEM((_NSEG,), jnp.float32),
            pltpu.VMEM((16, 128), jnp.float32),
            pltpu.VMEM((128,), jnp.float32),
            pltpu.VMEM_SHARED((16, _NSEG), jnp.float32),
        ],
    )
    def _segsum(s_hbm, b_hbm, out_hbm, vals, idx, acc, red, res, shared):
        cid = lax.axis_index("c")
        sid = lax.axis_index("s")
        wid = sid * 2 + cid
        base = wid * chunk

        zero = jnp.zeros((_L,), jnp.float32)

        def zbody(i, _):
            acc[pl.ds(i * _L, _L)] = zero
            return 0

        lax.fori_loop(0, _NSEG // _L, zbody, 0, unroll=8)

        is_last = wid == _NW - 1

        if last == chunk:
            pltpu.sync_copy(s_hbm.at[pl.ds(base, chunk)], vals)
            pltpu.sync_copy(b_hbm.at[pl.ds(batch_off + base, chunk)], idx)
        else:
            @pl.when(jnp.logical_not(is_last))
            def _():
                pltpu.sync_copy(s_hbm.at[pl.ds(base, chunk)], vals)
                pltpu.sync_copy(b_hbm.at[pl.ds(batch_off + base, chunk)], idx)

            @pl.when(is_last)
            def _():
                pltpu.sync_copy(s_hbm.at[pl.ds(base, last)],
                                vals.at[pl.ds(0, last)])
                pltpu.sync_copy(b_hbm.at[pl.ds(batch_off + base, last)],
                                idx.at[pl.ds(0, last)])

        def body(i, _):
            b = idx[pl.ds(i * _L, _L)]
            v = vals[pl.ds(i * _L, _L)]
            plsc.addupdate_scatter(acc, [b], v)
            return 0

        lax.fori_loop(0, last // _L, body, 0, unroll=4)

        if last != chunk:
            @pl.when(jnp.logical_not(is_last))
            def _():
                lax.fori_loop(last // _L, chunk // _L, body, 0, unroll=4)

        pltpu.sync_copy(acc, shared.at[sid])
        plsc.subcore_barrier()

        col = sid * 128
        pltpu.sync_copy(shared.at[:, pl.ds(col, 128)], red)
        for cchunk in range(128 // _L):
            v = red[0, pl.ds(cchunk * _L, _L)]
            for r in range(1, 16):
                v = v + red[r, pl.ds(cchunk * _L, _L)]
            res[pl.ds(cchunk * _L, _L)] = v

        pltpu.sync_copy(res, out_hbm.at[cid, pl.ds(col, 128)])

    return _segsum


_segsum1 = _make_segsum(_K1 // _NW, _K1 // _NW, 0)
_segsum2 = _make_segsum(1600, _K2 - 31 * 1600, _K1)


def kernel(energy, batch, W1, b1, W2, b2):
    s1 = _mlp(energy, W1, b1, W2, b2, _K1, 0)
    p1 = _segsum1(s1, batch)
    s2 = _mlp(energy, W1, b1, W2, b2, _K2, _K1 // _ROWS)
    p2 = _segsum2(s2, batch)
    return (p1[0] + p1[1]) + (p2[0] + p2[1])
